# baseline (speedup 1.0000x reference)
import jax
import jax.numpy as jnp
from jax import lax
from jax.experimental import pallas as pl
from jax.experimental.pallas import tpu as pltpu
from jax.experimental.pallas import tpu_sc as plsc

_NC = 2
_NS = 16
_NW = _NC * _NS


def _tc_select_body(vid_ref, sid_ref, ve_ref, se_ref, tok_ref, out_ref):
    vid = vid_ref[...]
    sid = sid_ref[...]
    ve = ve_ref[...]
    se = se_ref[...]
    vmask = (vid == 1).astype(jnp.float32)
    smask = (sid == 1).astype(jnp.float32)
    geom = (ve[0][None, :] + vmask * (ve[1] - ve[0])[None, :]
            + se[0][None, :] + smask * (se[1] - se[0])[None, :])
    out_ref[...] = tok_ref[...] + geom[:, None, :]


def _tc_geom_body(obuf_ref, geom_ref, tok_ref, out_ref):
    del obuf_ref
    out_ref[...] = tok_ref[...] + geom_ref[...][:, None, :]


def _make_sc_geom(b_sc, d, bpw):
    mesh = plsc.VectorSubcoreMesh(
        core_axis_name="c", subcore_axis_name="s",
        num_cores=_NC, num_subcores=_NS)

    def sc_geom(vids, sids, ctable):
        @pl.kernel(
            out_type=jax.ShapeDtypeStruct((b_sc, d), jnp.float32),
            mesh=mesh,
            scratch_types=[
                pltpu.VMEM((bpw,), jnp.int32),
                pltpu.VMEM((bpw,), jnp.int32),
                pltpu.VMEM((bpw,), jnp.int32),
                pltpu.VMEM((bpw, d), jnp.float32),
                pltpu.SemaphoreType.DMA,
            ],
        )
        def run(vids_hbm, sids_hbm, ctable_hbm, geom_hbm,
                vid_v, sid_v, c_v, rows_v, sem):
            wid = lax.axis_index("s") * _NC + lax.axis_index("c")
            base = wid * bpw
            pltpu.sync_copy(vids_hbm.at[pl.ds(base, bpw)], vid_v)
            pltpu.sync_copy(sids_hbm.at[pl.ds(base, bpw)], sid_v)
            for i in range(bpw // 16):
                s = pl.ds(i * 16, 16)
                c_v[s] = vid_v[s] * 2 + sid_v[s]
            pltpu.async_copy(ctable_hbm.at[c_v], rows_v, sem).wait()
            pltpu.sync_copy(rows_v, geom_hbm.at[pl.ds(base, bpw)])

        return run(vids, sids, ctable)

    return sc_geom


def kernel(tokens, view_ids, side_ids, view_embed, side_embed):
    B, L, D = tokens.shape
    BB = 128
    NB = B // BB
    NB1 = NB // 2
    B1 = NB1 * BB
    B2 = B - B1
    BPW = B2 // _NW

    vid2d = view_ids.astype(jnp.int32).reshape(B, 1)
    sid2d = side_ids.astype(jnp.int32).reshape(B, 1)

    ctable = (view_embed[:, None, :] + side_embed[None, :, :]).reshape(4, D)

    sc_geom = _make_sc_geom(B2, D, BPW)
    geom2 = sc_geom(view_ids[B1:].astype(jnp.int32),
                    side_ids[B1:].astype(jnp.int32), ctable)

    obuf = pl.pallas_call(
        _tc_select_body,
        grid=(NB1,),
        in_specs=[
            pl.BlockSpec((BB, 1), lambda i: (i, 0)),
            pl.BlockSpec((BB, 1), lambda i: (i, 0)),
            pl.BlockSpec((2, D), lambda i: (0, 0)),
            pl.BlockSpec((2, D), lambda i: (0, 0)),
            pl.BlockSpec((BB, L, D), lambda i: (i, 0, 0)),
        ],
        out_specs=pl.BlockSpec((BB, L, D), lambda i: (i, 0, 0)),
        out_shape=jax.ShapeDtypeStruct((B, L, D), tokens.dtype),
    )(vid2d, sid2d, view_embed, side_embed, tokens)

    out = pl.pallas_call(
        _tc_geom_body,
        grid=(NB - NB1,),
        in_specs=[
            pl.BlockSpec(memory_space=pl.ANY),
            pl.BlockSpec((BB, D), lambda i: (i, 0)),
            pl.BlockSpec((BB, L, D), lambda i: (i + NB1, 0, 0)),
        ],
        out_specs=pl.BlockSpec((BB, L, D), lambda i: (i + NB1, 0, 0)),
        out_shape=jax.ShapeDtypeStruct((B, L, D), tokens.dtype),
        input_output_aliases={0: 0},
    )(obuf, geom2, tokens)
    return out
